# Initial kernel scaffold; baseline (speedup 1.0000x reference)
import jax
import jax.numpy as jnp
from jax.experimental import pallas as pl


def kernel(x, edge_index_1, edge_weight_1, edge_index_2, edge_weight_2, edge_index_3, edge_weight_3, mapping_1, mapping_2, W1, b1, W2, b2, W3, b3, fc_w, fc_b):
    raise NotImplementedError("write your pallas kernel here")


# speedup vs baseline: 150.7385x; 150.7385x over previous
import functools

import jax
import jax.numpy as jnp
from jax import lax
from jax.experimental import pallas as pl
from jax.experimental.pallas import tpu as pltpu
from jax.experimental.pallas import tpu_sc as plsc

_F32 = jnp.float32
_NC = 2
_NS = 16


@functools.cache
def _adj_builder(n: int, e: int, blocks_per_sc: int):
    shard = e // _NS
    n_chunks = shard // 128
    block_rows = n // (_NC * blocks_per_sc)
    blk_words = block_rows * n
    z_per_tile = blk_words // _NS
    zwords = min(8192, z_per_tile)
    nz = z_per_tile // zwords
    degq = n // _NS
    swords = min(8192, z_per_tile)
    ns = z_per_tile // swords

    mesh = plsc.VectorSubcoreMesh(core_axis_name="c", subcore_axis_name="s")

    @functools.partial(
        pl.kernel,
        out_type=(
            jax.ShapeDtypeStruct((n * n,), _F32),
            jax.ShapeDtypeStruct((n,), _F32),
        ),
        mesh=mesh,
        scratch_types=[
            pltpu.VMEM((n_chunks, 128), jnp.int32),
            pltpu.VMEM((n_chunks, 128), jnp.int32),
            pltpu.VMEM((n_chunks, 128), _F32),
            pltpu.VMEM((n_chunks, 128), jnp.int32),
            pltpu.VMEM((n_chunks, 128), _F32),
            pltpu.VMEM((zwords,), _F32),
            pltpu.VMEM((swords,), _F32),
            pltpu.VMEM_SHARED((blk_words,), _F32),
            pltpu.VMEM_SHARED((n,), _F32),
        ],
    )
    def build(ei, ew, at_out, deg_out, r_buf, c_buf, w_buf, idx_buf, val_buf,
              zbuf, stage, sp_blk, deg_sp):
        cid = lax.axis_index("c")
        sid = lax.axis_index("s")
        base = sid * shard
        lane = lax.iota(jnp.int32, 16)

        for j in range(n_chunks):
            pltpu.sync_copy(ei.at[0, pl.ds(base + j * 128, 128)], r_buf.at[j])
            pltpu.sync_copy(ei.at[1, pl.ds(base + j * 128, 128)], c_buf.at[j])
            pltpu.sync_copy(ew.at[pl.ds(base + j * 128, 128)], w_buf.at[j])

        def _zb(i, carry):
            zbuf[pl.ds(i * 16, 16)] = jnp.zeros((16,), _F32)
            return carry
        lax.fori_loop(0, zwords // 16, _zb, 0)

        for b in range(blocks_per_sc):
            c0 = (cid * blocks_per_sc + b) * block_rows

            for z in range(nz):
                pltpu.sync_copy(
                    zbuf.at[pl.ds(0, zwords)],
                    sp_blk.at[pl.ds(sid * z_per_tile + z * zwords, zwords)])
            if b == 0:
                @pl.when(cid == 0)
                def _():
                    pltpu.sync_copy(zbuf.at[pl.ds(0, degq)],
                                    deg_sp.at[pl.ds(sid * degq, degq)])
            plsc.subcore_barrier()

            def _chunk(j, carry):
                for g in range(8):
                    sl = pl.ds(g * 16, 16)
                    r = r_buf[j, sl]
                    c = c_buf[j, sl]
                    w = w_buf[j, sl]
                    m = (c >= c0) & (c < c0 + block_rows)
                    fi = (c - c0) * n + r
                    pad = ((j * 128 + g * 16) + lane) * _NS + sid
                    idx_buf[j, sl] = jnp.where(m, fi, pad)
                    val_buf[j, sl] = jnp.where(m, w, 0.0)
                pltpu.sync_copy(val_buf.at[j], sp_blk.at[idx_buf.at[j]], add=True)
                return carry
            lax.fori_loop(0, n_chunks, _chunk, 0)

            if b == 0:
                @pl.when(cid == 0)
                def _():
                    def _dchunk(j, carry):
                        pltpu.sync_copy(w_buf.at[j], deg_sp.at[r_buf.at[j]],
                                        add=True)
                        return carry
                    lax.fori_loop(0, n_chunks, _dchunk, 0)
            plsc.subcore_barrier()

            if b == 0:
                @pl.when(cid == 0)
                def _():
                    pltpu.sync_copy(deg_sp.at[pl.ds(sid * degq, degq)],
                                    stage.at[pl.ds(0, degq)])
                    pltpu.sync_copy(stage.at[pl.ds(0, degq)],
                                    deg_out.at[pl.ds(sid * degq, degq)])
            for s in range(ns):
                off = sid * z_per_tile + s * swords
                pltpu.sync_copy(sp_blk.at[pl.ds(off, swords)], stage)
                pltpu.sync_copy(stage, at_out.at[pl.ds(c0 * n + off, swords)])
            plsc.subcore_barrier()

    return build


@functools.cache
def _cheb(n: int, r: int, k_order: int):
    def body(x_ref, at_ref, deg_ref, out_ref, tm1, tm2, dis):
        k = pl.program_id(0)

        @pl.when(k == 0)
        def _():
            d = deg_ref[...]
            dis[...] = jnp.where(d > 0, lax.rsqrt(jnp.where(d > 0, d, 1.0)), 0.0)
            t0 = x_ref[...]
            tm1[...] = t0
            out_ref[...] = t0[None]

        @pl.when(k == 1)
        def _():
            t = -jnp.dot(tm1[...] * dis[...], at_ref[...],
                         preferred_element_type=_F32, precision=lax.Precision.HIGHEST) * dis[...]
            tm2[...] = tm1[...]
            tm1[...] = t
            out_ref[...] = t[None]

        @pl.when(k >= 2)
        def _():
            t = (-2.0 * jnp.dot(tm1[...] * dis[...], at_ref[...],
                                preferred_element_type=_F32, precision=lax.Precision.HIGHEST) * dis[...]
                 - tm2[...])
            tm2[...] = tm1[...]
            tm1[...] = t
            out_ref[...] = t[None]

    return pl.pallas_call(
        body,
        grid=(k_order,),
        in_specs=[
            pl.BlockSpec((r, n), lambda k: (0, 0)),
            pl.BlockSpec((n, n), lambda k: (0, 0)),
            pl.BlockSpec((1, n), lambda k: (0, 0)),
        ],
        out_specs=pl.BlockSpec((1, r, n), lambda k: (k, 0, 0)),
        out_shape=jax.ShapeDtypeStruct((k_order, r, n), _F32),
        scratch_shapes=[
            pltpu.VMEM((r, n), _F32),
            pltpu.VMEM((r, n), _F32),
            pltpu.VMEM((1, n), _F32),
        ],
    )


@functools.cache
def _contract(n: int, k_order: int, i_ch: int, o_ch: int, h_t: int, b_n: int,
              relu: bool):
    ki = k_order * i_ch

    def body(t_ref, w_ref, b_ref, out_ref):
        xk = t_ref[...].reshape(ki, n)
        y = jnp.dot(w_ref[0], xk, preferred_element_type=_F32, precision=lax.Precision.HIGHEST) + b_ref[...]
        if relu:
            y = jnp.maximum(y, 0.0)
        out_ref[...] = y.reshape(1, 1, o_ch, n)

    return pl.pallas_call(
        body,
        grid=(b_n, h_t),
        in_specs=[
            pl.BlockSpec((k_order, 1, 1, i_ch, n), lambda b, h: (0, h, b, 0, 0)),
            pl.BlockSpec((1, o_ch, ki), lambda b, h: (h, 0, 0)),
            pl.BlockSpec((o_ch, 1), lambda b, h: (0, 0)),
        ],
        out_specs=pl.BlockSpec((1, 1, o_ch, n), lambda b, h: (h, b, 0, 0)),
        out_shape=jax.ShapeDtypeStruct((h_t, b_n, o_ch, n), _F32),
    )


@functools.cache
def _contract_collapse(n: int, k_order: int, i_ch: int, o_ch: int, h_t: int,
                       b_n: int):
    ki = k_order * i_ch

    def body(t_ref, w_ref, b_ref, out_ref):
        h = pl.program_id(1)
        xk = t_ref[...].reshape(ki, n)
        y = jnp.dot(w_ref[0], xk, preferred_element_type=_F32, precision=lax.Precision.HIGHEST)

        @pl.when(h == 0)
        def _():
            out_ref[...] = (y + float(h_t) * b_ref[...])[None]

        @pl.when(h > 0)
        def _():
            out_ref[...] = out_ref[...] + y[None]

    return pl.pallas_call(
        body,
        grid=(b_n, h_t),
        in_specs=[
            pl.BlockSpec((k_order, 1, 1, i_ch, n), lambda b, h: (0, h, b, 0, 0)),
            pl.BlockSpec((1, o_ch, ki), lambda b, h: (h, 0, 0)),
            pl.BlockSpec((o_ch, 1), lambda b, h: (0, 0)),
        ],
        out_specs=pl.BlockSpec((1, o_ch, n), lambda b, h: (b, 0, 0)),
        out_shape=jax.ShapeDtypeStruct((b_n, o_ch, n), _F32),
    )


@functools.cache
def _pool(m_rows: int, n_in: int, n_out: int):
    def body(a_ref, m_ref, o_ref):
        o_ref[...] = lax.dot_general(
            a_ref[...], m_ref[...], (((1,), (1,)), ((), ())),
            preferred_element_type=_F32, precision=lax.Precision.HIGHEST)

    return pl.pallas_call(
        body,
        in_specs=[
            pl.BlockSpec((m_rows, n_in), lambda: (0, 0)),
            pl.BlockSpec((n_out, n_in), lambda: (0, 0)),
        ],
        out_specs=pl.BlockSpec((m_rows, n_out), lambda: (0, 0)),
        out_shape=jax.ShapeDtypeStruct((m_rows, n_out), _F32),
    )


@functools.cache
def _fc(b_n: int, feat: int, n_cls: int):
    def body(z_ref, w_ref, b_ref, o_ref):
        logits = lax.dot_general(
            z_ref[...], w_ref[...], (((1,), (1,)), ((), ())),
            preferred_element_type=_F32, precision=lax.Precision.HIGHEST) + b_ref[...]
        m = jnp.max(logits, axis=1, keepdims=True)
        zz = logits - m
        o_ref[...] = zz - jnp.log(jnp.sum(jnp.exp(zz), axis=1, keepdims=True))

    return pl.pallas_call(
        body,
        in_specs=[
            pl.BlockSpec((b_n, feat), lambda: (0, 0)),
            pl.BlockSpec((n_cls, feat), lambda: (0, 0)),
            pl.BlockSpec((1, n_cls), lambda: (0, 0)),
        ],
        out_specs=pl.BlockSpec((b_n, n_cls), lambda: (0, 0)),
        out_shape=jax.ShapeDtypeStruct((b_n, n_cls), _F32),
    )


def kernel(x, edge_index_1, edge_weight_1, edge_index_2, edge_weight_2,
           edge_index_3, edge_weight_3, mapping_1, mapping_2, W1, b1, W2, b2,
           W3, b3, fc_w, fc_b):
    b_n, n1, h_t = x.shape
    k_order = W1.shape[0]
    n2, n3 = mapping_1.shape[0], mapping_2.shape[0]
    g1, g2, g3 = W1.shape[3], W2.shape[3], W3.shape[3]
    n_cls = fc_w.shape[0]

    at1, deg1 = _adj_builder(n1, edge_index_1.shape[1], 2)(edge_index_1,
                                                           edge_weight_1)
    at2, deg2 = _adj_builder(n2, edge_index_2.shape[1], 1)(edge_index_2,
                                                           edge_weight_2)
    at3, deg3 = _adj_builder(n3, edge_index_3.shape[1], 1)(edge_index_3,
                                                           edge_weight_3)

    xt1 = jnp.transpose(x, (2, 0, 1)).reshape(h_t * b_n, n1)
    tall1 = _cheb(n1, h_t * b_n, k_order)(
        xt1, at1.reshape(n1, n1), deg1.reshape(1, n1))
    wt1 = jnp.transpose(W1, (1, 3, 0, 2)).reshape(h_t, g1, k_order)
    o1 = _contract(n1, k_order, 1, g1, h_t, b_n, True)(
        tall1.reshape(k_order, h_t, b_n, 1, n1), wt1, b1.reshape(g1, 1))
    x2 = _pool(h_t * b_n * g1, n1, n2)(o1.reshape(h_t * b_n * g1, n1),
                                       mapping_1)

    tall2 = _cheb(n2, h_t * b_n * g1, k_order)(
        x2.reshape(h_t * b_n * g1, n2), at2.reshape(n2, n2),
        deg2.reshape(1, n2))
    wt2 = jnp.transpose(W2, (1, 3, 0, 2)).reshape(h_t, g2, k_order * g1)
    o2 = _contract(n2, k_order, g1, g2, h_t, b_n, True)(
        tall2.reshape(k_order, h_t, b_n, g1, n2), wt2, b2.reshape(g2, 1))
    x3 = _pool(h_t * b_n * g2, n2, n3)(o2.reshape(h_t * b_n * g2, n2),
                                       mapping_2)

    tall3 = _cheb(n3, h_t * b_n * g2, k_order)(
        x3.reshape(h_t * b_n * g2, n3), at3.reshape(n3, n3),
        deg3.reshape(1, n3))
    wt3 = jnp.transpose(W3, (1, 3, 0, 2)).reshape(h_t, g3, k_order * g2)
    y = _contract_collapse(n3, k_order, g2, g3, h_t, b_n)(
        tall3.reshape(k_order, h_t, b_n, g2, n3), wt3, b3.reshape(g3, 1))

    z = jnp.transpose(y, (2, 1, 0)).reshape(b_n, n3 * g3)
    return _fc(b_n, n3 * g3, n_cls)(z, fc_w, fc_b.reshape(1, n_cls))


# speedup vs baseline: 182.8163x; 1.2128x over previous
import functools

import jax
import jax.numpy as jnp
from jax import lax
from jax.experimental import pallas as pl
from jax.experimental.pallas import tpu as pltpu
from jax.experimental.pallas import tpu_sc as plsc

_F32 = jnp.float32
_NC = 2
_NS = 16


@functools.cache
def _adj_builder(n: int, e: int, blocks_per_sc: int):
    shard = e // _NS
    n_chunks = shard // 128
    block_rows = n // (_NC * blocks_per_sc)
    blk_words = block_rows * n
    z_per_tile = blk_words // _NS
    zwords = min(8192, z_per_tile)
    nz = z_per_tile // zwords
    degq = n // _NS
    swords = min(8192, z_per_tile)
    ns = z_per_tile // swords

    mesh = plsc.VectorSubcoreMesh(core_axis_name="c", subcore_axis_name="s")

    @functools.partial(
        pl.kernel,
        out_type=(
            jax.ShapeDtypeStruct((n * n,), _F32),
            jax.ShapeDtypeStruct((n,), _F32),
        ),
        mesh=mesh,
        scratch_types=[
            pltpu.VMEM((n_chunks, 128), jnp.int32),
            pltpu.VMEM((n_chunks, 128), jnp.int32),
            pltpu.VMEM((n_chunks, 128), _F32),
            pltpu.VMEM((n_chunks, 128), jnp.int32),
            pltpu.VMEM((n_chunks, 128), _F32),
            pltpu.VMEM((zwords,), _F32),
            pltpu.VMEM((swords,), _F32),
            pltpu.VMEM_SHARED((blk_words,), _F32),
            pltpu.VMEM_SHARED((n,), _F32),
        ],
    )
    def build(ei, ew, at_out, deg_out, r_buf, c_buf, w_buf, idx_buf, val_buf,
              zbuf, stage, sp_blk, deg_sp):
        cid = lax.axis_index("c")
        sid = lax.axis_index("s")
        base = sid * shard
        lane = lax.iota(jnp.int32, 16)

        for j in range(n_chunks):
            pltpu.sync_copy(ei.at[0, pl.ds(base + j * 128, 128)], r_buf.at[j])
            pltpu.sync_copy(ei.at[1, pl.ds(base + j * 128, 128)], c_buf.at[j])
            pltpu.sync_copy(ew.at[pl.ds(base + j * 128, 128)], w_buf.at[j])

        def _zb(i, carry):
            zbuf[pl.ds(i * 16, 16)] = jnp.zeros((16,), _F32)
            return carry
        lax.fori_loop(0, zwords // 16, _zb, 0)

        for b in range(blocks_per_sc):
            c0 = (cid * blocks_per_sc + b) * block_rows

            for z in range(nz):
                pltpu.sync_copy(
                    zbuf.at[pl.ds(0, zwords)],
                    sp_blk.at[pl.ds(sid * z_per_tile + z * zwords, zwords)])
            if b == 0:
                @pl.when(cid == 0)
                def _():
                    pltpu.sync_copy(zbuf.at[pl.ds(0, degq)],
                                    deg_sp.at[pl.ds(sid * degq, degq)])
            plsc.subcore_barrier()

            def _chunk(j, carry):
                for g in range(8):
                    sl = pl.ds(g * 16, 16)
                    r = r_buf[j, sl]
                    c = c_buf[j, sl]
                    w = w_buf[j, sl]
                    m = (c >= c0) & (c < c0 + block_rows)
                    fi = (c - c0) * n + r
                    pad = ((j * 128 + g * 16) + lane) * _NS + sid
                    idx_buf[j, sl] = jnp.where(m, fi, pad)
                    val_buf[j, sl] = jnp.where(m, w, 0.0)
                pltpu.sync_copy(val_buf.at[j], sp_blk.at[idx_buf.at[j]], add=True)
                return carry
            lax.fori_loop(0, n_chunks, _chunk, 0)

            if b == 0:
                @pl.when(cid == 0)
                def _():
                    def _dchunk(j, carry):
                        pltpu.sync_copy(w_buf.at[j], deg_sp.at[r_buf.at[j]],
                                        add=True)
                        return carry
                    lax.fori_loop(0, n_chunks, _dchunk, 0)
            plsc.subcore_barrier()

            if b == 0:
                @pl.when(cid == 0)
                def _():
                    pltpu.sync_copy(deg_sp.at[pl.ds(sid * degq, degq)],
                                    stage.at[pl.ds(0, degq)])
                    pltpu.sync_copy(stage.at[pl.ds(0, degq)],
                                    deg_out.at[pl.ds(sid * degq, degq)])
            for s in range(ns):
                off = sid * z_per_tile + s * swords
                pltpu.sync_copy(sp_blk.at[pl.ds(off, swords)], stage)
                pltpu.sync_copy(stage, at_out.at[pl.ds(c0 * n + off, swords)])
            plsc.subcore_barrier()

    return build


_BF = jnp.bfloat16


def _bsplit(v):
    hi = v.astype(_BF)
    lo = (v - hi.astype(_F32)).astype(_BF)
    return hi, lo


@functools.cache
def _cheb(n: int, r: int, k_order: int):
    def body(x_ref, at_ref, deg_ref, hi_ref, lo_ref, tm1, tm2, dis, ah, al):
        k = pl.program_id(0)

        @pl.when(k == 0)
        def _():
            d = deg_ref[...]
            dis[...] = jnp.where(d > 0, lax.rsqrt(jnp.where(d > 0, d, 1.0)), 0.0)
            a_hi, a_lo = _bsplit(at_ref[...])
            ah[...] = a_hi
            al[...] = a_lo
            t0 = x_ref[...]
            tm1[...] = t0
            tm2[...] = jnp.zeros_like(t0)
            h0, l0 = _bsplit(t0)
            hi_ref[...] = h0[None]
            lo_ref[...] = l0[None]

        @pl.when(k >= 1)
        def _():
            u = tm1[...] * dis[...]
            uh, ul = _bsplit(u)
            p = (jnp.dot(uh, ah[...], preferred_element_type=_F32)
                 + jnp.dot(ul, ah[...], preferred_element_type=_F32)
                 + jnp.dot(uh, al[...], preferred_element_type=_F32))
            coef = jnp.where(k == 1, -1.0, -2.0).astype(_F32)
            t = coef * (p * dis[...]) - tm2[...]
            tm2[...] = tm1[...]
            tm1[...] = t
            th, tl = _bsplit(t)
            hi_ref[...] = th[None]
            lo_ref[...] = tl[None]

    return pl.pallas_call(
        body,
        grid=(k_order,),
        in_specs=[
            pl.BlockSpec((r, n), lambda k: (0, 0)),
            pl.BlockSpec((n, n), lambda k: (0, 0)),
            pl.BlockSpec((1, n), lambda k: (0, 0)),
        ],
        out_specs=[
            pl.BlockSpec((1, r, n), lambda k: (k, 0, 0)),
            pl.BlockSpec((1, r, n), lambda k: (k, 0, 0)),
        ],
        out_shape=[
            jax.ShapeDtypeStruct((k_order, r, n), _BF),
            jax.ShapeDtypeStruct((k_order, r, n), _BF),
        ],
        scratch_shapes=[
            pltpu.VMEM((r, n), _F32),
            pltpu.VMEM((r, n), _F32),
            pltpu.VMEM((1, n), _F32),
            pltpu.VMEM((n, n), _BF),
            pltpu.VMEM((n, n), _BF),
        ],
    )


@functools.cache
def _contract(n: int, k_order: int, i_ch: int, o_ch: int, h_t: int, b_n: int,
              relu: bool):
    ki = k_order * i_ch

    def body(th_ref, tl_ref, wh_ref, wl_ref, b_ref, out_ref):
        xh = th_ref[...].reshape(ki, n)
        xl = tl_ref[...].reshape(ki, n)
        wh = wh_ref[0]
        wl = wl_ref[0]
        y = (jnp.dot(wh, xh, preferred_element_type=_F32)
             + jnp.dot(wl, xh, preferred_element_type=_F32)
             + jnp.dot(wh, xl, preferred_element_type=_F32)) + b_ref[...]
        if relu:
            y = jnp.maximum(y, 0.0)
        out_ref[...] = y.reshape(1, 1, o_ch, n)

    return pl.pallas_call(
        body,
        grid=(b_n, h_t),
        in_specs=[
            pl.BlockSpec((k_order, 1, 1, i_ch, n), lambda b, h: (0, h, b, 0, 0)),
            pl.BlockSpec((k_order, 1, 1, i_ch, n), lambda b, h: (0, h, b, 0, 0)),
            pl.BlockSpec((1, o_ch, ki), lambda b, h: (h, 0, 0)),
            pl.BlockSpec((1, o_ch, ki), lambda b, h: (h, 0, 0)),
            pl.BlockSpec((o_ch, 1), lambda b, h: (0, 0)),
        ],
        out_specs=pl.BlockSpec((1, 1, o_ch, n), lambda b, h: (h, b, 0, 0)),
        out_shape=jax.ShapeDtypeStruct((h_t, b_n, o_ch, n), _F32),
    )


@functools.cache
def _contract_collapse(n: int, k_order: int, i_ch: int, o_ch: int, h_t: int,
                       b_n: int):
    ki = k_order * i_ch

    def body(th_ref, tl_ref, wh_ref, wl_ref, b_ref, out_ref):
        h = pl.program_id(1)
        xh = th_ref[...].reshape(ki, n)
        xl = tl_ref[...].reshape(ki, n)
        wh = wh_ref[0]
        wl = wl_ref[0]
        y = (jnp.dot(wh, xh, preferred_element_type=_F32)
             + jnp.dot(wl, xh, preferred_element_type=_F32)
             + jnp.dot(wh, xl, preferred_element_type=_F32))

        @pl.when(h == 0)
        def _():
            out_ref[...] = (y + float(h_t) * b_ref[...])[None]

        @pl.when(h > 0)
        def _():
            out_ref[...] = out_ref[...] + y[None]

    return pl.pallas_call(
        body,
        grid=(b_n, h_t),
        in_specs=[
            pl.BlockSpec((k_order, 1, 1, i_ch, n), lambda b, h: (0, h, b, 0, 0)),
            pl.BlockSpec((k_order, 1, 1, i_ch, n), lambda b, h: (0, h, b, 0, 0)),
            pl.BlockSpec((1, o_ch, ki), lambda b, h: (h, 0, 0)),
            pl.BlockSpec((1, o_ch, ki), lambda b, h: (h, 0, 0)),
            pl.BlockSpec((o_ch, 1), lambda b, h: (0, 0)),
        ],
        out_specs=pl.BlockSpec((1, o_ch, n), lambda b, h: (b, 0, 0)),
        out_shape=jax.ShapeDtypeStruct((b_n, o_ch, n), _F32),
    )


@functools.cache
def _pool(m_rows: int, n_in: int, n_out: int):
    dn = (((1,), (1,)), ((), ()))

    def body(a_ref, m_ref, o_ref):
        ah, al = _bsplit(a_ref[...])
        mh, ml = _bsplit(m_ref[...])
        o_ref[...] = (lax.dot_general(ah, mh, dn, preferred_element_type=_F32)
                      + lax.dot_general(al, mh, dn, preferred_element_type=_F32)
                      + lax.dot_general(ah, ml, dn, preferred_element_type=_F32))

    return pl.pallas_call(
        body,
        in_specs=[
            pl.BlockSpec((m_rows, n_in), lambda: (0, 0)),
            pl.BlockSpec((n_out, n_in), lambda: (0, 0)),
        ],
        out_specs=pl.BlockSpec((m_rows, n_out), lambda: (0, 0)),
        out_shape=jax.ShapeDtypeStruct((m_rows, n_out), _F32),
    )


@functools.cache
def _fc(b_n: int, feat: int, n_cls: int):
    dn = (((1,), (1,)), ((), ()))

    def body(z_ref, w_ref, b_ref, o_ref):
        zh, zl = _bsplit(z_ref[...])
        wh, wl = _bsplit(w_ref[...])
        logits = (lax.dot_general(zh, wh, dn, preferred_element_type=_F32)
                  + lax.dot_general(zl, wh, dn, preferred_element_type=_F32)
                  + lax.dot_general(zh, wl, dn, preferred_element_type=_F32)
                  ) + b_ref[...]
        m = jnp.max(logits, axis=1, keepdims=True)
        zz = logits - m
        o_ref[...] = zz - jnp.log(jnp.sum(jnp.exp(zz), axis=1, keepdims=True))

    return pl.pallas_call(
        body,
        in_specs=[
            pl.BlockSpec((b_n, feat), lambda: (0, 0)),
            pl.BlockSpec((n_cls, feat), lambda: (0, 0)),
            pl.BlockSpec((1, n_cls), lambda: (0, 0)),
        ],
        out_specs=pl.BlockSpec((b_n, n_cls), lambda: (0, 0)),
        out_shape=jax.ShapeDtypeStruct((b_n, n_cls), _F32),
    )


def kernel(x, edge_index_1, edge_weight_1, edge_index_2, edge_weight_2,
           edge_index_3, edge_weight_3, mapping_1, mapping_2, W1, b1, W2, b2,
           W3, b3, fc_w, fc_b):
    b_n, n1, h_t = x.shape
    k_order = W1.shape[0]
    n2, n3 = mapping_1.shape[0], mapping_2.shape[0]
    g1, g2, g3 = W1.shape[3], W2.shape[3], W3.shape[3]
    n_cls = fc_w.shape[0]

    at1, deg1 = _adj_builder(n1, edge_index_1.shape[1], 2)(edge_index_1,
                                                           edge_weight_1)
    at2, deg2 = _adj_builder(n2, edge_index_2.shape[1], 1)(edge_index_2,
                                                           edge_weight_2)
    at3, deg3 = _adj_builder(n3, edge_index_3.shape[1], 1)(edge_index_3,
                                                           edge_weight_3)

    xt1 = jnp.transpose(x, (2, 0, 1)).reshape(h_t * b_n, n1)
    th1, tl1 = _cheb(n1, h_t * b_n, k_order)(
        xt1, at1.reshape(n1, n1), deg1.reshape(1, n1))
    wt1h, wt1l = _bsplit(jnp.transpose(W1, (1, 3, 0, 2)).reshape(h_t, g1,
                                                                 k_order))
    o1 = _contract(n1, k_order, 1, g1, h_t, b_n, True)(
        th1.reshape(k_order, h_t, b_n, 1, n1),
        tl1.reshape(k_order, h_t, b_n, 1, n1), wt1h, wt1l, b1.reshape(g1, 1))
    x2 = _pool(h_t * b_n * g1, n1, n2)(o1.reshape(h_t * b_n * g1, n1),
                                       mapping_1)

    th2, tl2 = _cheb(n2, h_t * b_n * g1, k_order)(
        x2.reshape(h_t * b_n * g1, n2), at2.reshape(n2, n2),
        deg2.reshape(1, n2))
    wt2h, wt2l = _bsplit(jnp.transpose(W2, (1, 3, 0, 2)).reshape(
        h_t, g2, k_order * g1))
    o2 = _contract(n2, k_order, g1, g2, h_t, b_n, True)(
        th2.reshape(k_order, h_t, b_n, g1, n2),
        tl2.reshape(k_order, h_t, b_n, g1, n2), wt2h, wt2l, b2.reshape(g2, 1))
    x3 = _pool(h_t * b_n * g2, n2, n3)(o2.reshape(h_t * b_n * g2, n2),
                                       mapping_2)

    th3, tl3 = _cheb(n3, h_t * b_n * g2, k_order)(
        x3.reshape(h_t * b_n * g2, n3), at3.reshape(n3, n3),
        deg3.reshape(1, n3))
    wt3h, wt3l = _bsplit(jnp.transpose(W3, (1, 3, 0, 2)).reshape(
        h_t, g3, k_order * g2))
    y = _contract_collapse(n3, k_order, g2, g3, h_t, b_n)(
        th3.reshape(k_order, h_t, b_n, g2, n3),
        tl3.reshape(k_order, h_t, b_n, g2, n3), wt3h, wt3l, b3.reshape(g3, 1))

    z = jnp.transpose(y, (2, 1, 0)).reshape(b_n, n3 * g3)
    return _fc(b_n, n3 * g3, n_cls)(z, fc_w, fc_b.reshape(1, n_cls))


# speedup vs baseline: 197.9554x; 1.0828x over previous
import functools

import jax
import jax.numpy as jnp
from jax import lax
from jax.experimental import pallas as pl
from jax.experimental.pallas import tpu as pltpu
from jax.experimental.pallas import tpu_sc as plsc

_F32 = jnp.float32
_NC = 2
_NS = 16


@functools.cache
def _adj_builder(n: int, e: int, blocks_per_sc: int):
    shard = e // _NS
    n_chunks = shard // 128
    block_rows = n // (_NC * blocks_per_sc)
    blk_words = block_rows * n
    z_per_tile = blk_words // _NS
    zwords = min(8192, z_per_tile)
    nz = z_per_tile // zwords
    degq = n // _NS
    swords = min(8192, z_per_tile)
    ns = z_per_tile // swords

    mesh = plsc.VectorSubcoreMesh(core_axis_name="c", subcore_axis_name="s")

    @functools.partial(
        pl.kernel,
        out_type=(
            jax.ShapeDtypeStruct((n * n,), _F32),
            jax.ShapeDtypeStruct((n,), _F32),
        ),
        mesh=mesh,
        scratch_types=[
            pltpu.VMEM((shard,), jnp.int32),
            pltpu.VMEM((shard,), jnp.int32),
            pltpu.VMEM((shard,), _F32),
            pltpu.VMEM((n_chunks, 128), jnp.int32),
            pltpu.VMEM((n_chunks, 128), _F32),
            pltpu.VMEM((n_chunks, 128), jnp.int32),
            pltpu.VMEM((n_chunks, 128), _F32),
            pltpu.VMEM((zwords,), _F32),
            pltpu.VMEM((swords,), _F32),
            pltpu.VMEM_SHARED((blk_words,), _F32),
            pltpu.VMEM_SHARED((n,), _F32),
            pltpu.SemaphoreType.DMA,
        ],
    )
    def build(ei, ew, at_out, deg_out, r_buf, c_buf, w_buf, idx_buf, val_buf,
              didx_buf, dval_buf, zbuf, stage, sp_blk, deg_sp, sem):
        cid = lax.axis_index("c")
        sid = lax.axis_index("s")
        base = sid * shard
        lane = lax.iota(jnp.int32, 16)

        pltpu.async_copy(ei.at[0, pl.ds(base, shard)], r_buf, sem)
        pltpu.async_copy(ei.at[1, pl.ds(base, shard)], c_buf, sem)
        cw = pltpu.async_copy(ew.at[pl.ds(base, shard)], w_buf, sem)

        def _zb(i, carry):
            zbuf[pl.ds(i * 16, 16)] = jnp.zeros((16,), _F32)
            return carry
        lax.fori_loop(0, zwords // 16, _zb, 0)
        cw.wait()
        cw.wait()
        cw.wait()

        for b in range(blocks_per_sc):
            c0 = (cid * blocks_per_sc + b) * block_rows

            zd = [pltpu.async_copy(
                zbuf.at[pl.ds(0, zwords)],
                sp_blk.at[pl.ds(sid * z_per_tile + z * zwords, zwords)], sem)
                for z in range(nz)]
            if b == 0:
                @pl.when(cid == 0)
                def _():
                    pltpu.sync_copy(zbuf.at[pl.ds(0, degq)],
                                    deg_sp.at[pl.ds(sid * degq, degq)])
            for z in zd:
                z.wait()
            plsc.subcore_barrier()

            def _scan(j, carry):
                for g in range(8):
                    p0 = j * 128 + g * 16
                    sl = pl.ds(g * 16, 16)
                    psl = pl.ds(p0, 16)
                    r = r_buf[psl]
                    c = c_buf[psl]
                    w = w_buf[psl]
                    m = (c >= c0) & (c < c0 + block_rows)
                    fi = (c - c0) * n + r
                    pad = (p0 + lane) * _NS + sid
                    idx_buf[j, sl] = jnp.where(m, fi, pad)
                    val_buf[j, sl] = jnp.where(m, w, 0.0)
                    if b == 0:
                        didx_buf[j, sl] = r
                        dval_buf[j, sl] = w
                return carry
            lax.fori_loop(0, n_chunks, _scan, 0)

            sc_d = [pltpu.async_copy(val_buf.at[j], sp_blk.at[idx_buf.at[j]],
                                     sem, add=True) for j in range(n_chunks)]
            if b == 0:
                @pl.when(cid == 0)
                def _():
                    def _dchunk(j, carry):
                        pltpu.sync_copy(dval_buf.at[j],
                                        deg_sp.at[didx_buf.at[j]], add=True)
                        return carry
                    lax.fori_loop(0, n_chunks, _dchunk, 0)
            for d in sc_d:
                d.wait()
            plsc.subcore_barrier()

            if b == 0:
                @pl.when(cid == 0)
                def _():
                    pltpu.sync_copy(deg_sp.at[pl.ds(sid * degq, degq)],
                                    stage.at[pl.ds(0, degq)])
                    pltpu.sync_copy(stage.at[pl.ds(0, degq)],
                                    deg_out.at[pl.ds(sid * degq, degq)])
            for t in range(ns):
                off = sid * z_per_tile + t * swords
                pltpu.sync_copy(sp_blk.at[pl.ds(off, swords)], stage)
                pltpu.sync_copy(stage, at_out.at[pl.ds(c0 * n + off, swords)])
            plsc.subcore_barrier()

    return build


_BF = jnp.bfloat16


def _bsplit(v):
    hi = v.astype(_BF)
    lo = (v - hi.astype(_F32)).astype(_BF)
    return hi, lo


@functools.cache
def _cheb(n: int, r: int, k_order: int):
    def _dot3(uh, ul, ah, al):
        return (jnp.dot(uh, ah[...], preferred_element_type=_F32)
                + jnp.dot(ul, ah[...], preferred_element_type=_F32)
                + jnp.dot(uh, al[...], preferred_element_type=_F32))

    def body(x_ref, at_ref, deg_ref, hi_ref, lo_ref, ta, tb, dis, ah, al):
        k = pl.program_id(0)

        @pl.when(k == 0)
        def _():
            d = deg_ref[...]
            dis[...] = jnp.where(d > 0, lax.rsqrt(jnp.where(d > 0, d, 1.0)), 0.0)
            a_hi, a_lo = _bsplit(at_ref[...])
            ah[...] = a_hi
            al[...] = a_lo
            t0 = x_ref[...]
            ta[...] = t0
            tb[...] = jnp.zeros_like(t0)
            h0, l0 = _bsplit(t0)
            hi_ref[...] = h0[None]
            lo_ref[...] = l0[None]

        @pl.when((k % 2) == 1)
        def _():
            u = ta[...] * dis[...]
            uh, ul = _bsplit(u)
            p = _dot3(uh, ul, ah, al)
            coef = jnp.where(k == 1, -1.0, -2.0).astype(_F32)
            t = coef * (p * dis[...]) - tb[...]
            tb[...] = t
            th, tl = _bsplit(t)
            hi_ref[...] = th[None]
            lo_ref[...] = tl[None]

        @pl.when((k >= 2) & ((k % 2) == 0))
        def _():
            u = tb[...] * dis[...]
            uh, ul = _bsplit(u)
            p = _dot3(uh, ul, ah, al)
            t = -2.0 * (p * dis[...]) - ta[...]
            ta[...] = t
            th, tl = _bsplit(t)
            hi_ref[...] = th[None]
            lo_ref[...] = tl[None]

    return pl.pallas_call(
        body,
        grid=(k_order,),
        in_specs=[
            pl.BlockSpec((r, n), lambda k: (0, 0)),
            pl.BlockSpec((n, n), lambda k: (0, 0)),
            pl.BlockSpec((1, n), lambda k: (0, 0)),
        ],
        out_specs=[
            pl.BlockSpec((1, r, n), lambda k: (k, 0, 0)),
            pl.BlockSpec((1, r, n), lambda k: (k, 0, 0)),
        ],
        out_shape=[
            jax.ShapeDtypeStruct((k_order, r, n), _BF),
            jax.ShapeDtypeStruct((k_order, r, n), _BF),
        ],
        scratch_shapes=[
            pltpu.VMEM((r, n), _F32),
            pltpu.VMEM((r, n), _F32),
            pltpu.VMEM((1, n), _F32),
            pltpu.VMEM((n, n), _BF),
            pltpu.VMEM((n, n), _BF),
        ],
    )


@functools.cache
def _contract(n: int, k_order: int, i_ch: int, o_ch: int, h_t: int, b_n: int,
              relu: bool):
    ki = k_order * i_ch

    def body(th_ref, tl_ref, wh_ref, wl_ref, b_ref, out_ref):
        xh = th_ref[...].reshape(ki, n)
        xl = tl_ref[...].reshape(ki, n)
        wh = wh_ref[0]
        wl = wl_ref[0]
        y = (jnp.dot(wh, xh, preferred_element_type=_F32)
             + jnp.dot(wl, xh, preferred_element_type=_F32)
             + jnp.dot(wh, xl, preferred_element_type=_F32)) + b_ref[...]
        if relu:
            y = jnp.maximum(y, 0.0)
        out_ref[...] = y.reshape(1, 1, o_ch, n)

    return pl.pallas_call(
        body,
        grid=(b_n, h_t),
        in_specs=[
            pl.BlockSpec((k_order, 1, 1, i_ch, n), lambda b, h: (0, h, b, 0, 0)),
            pl.BlockSpec((k_order, 1, 1, i_ch, n), lambda b, h: (0, h, b, 0, 0)),
            pl.BlockSpec((1, o_ch, ki), lambda b, h: (h, 0, 0)),
            pl.BlockSpec((1, o_ch, ki), lambda b, h: (h, 0, 0)),
            pl.BlockSpec((o_ch, 1), lambda b, h: (0, 0)),
        ],
        out_specs=pl.BlockSpec((1, 1, o_ch, n), lambda b, h: (h, b, 0, 0)),
        out_shape=jax.ShapeDtypeStruct((h_t, b_n, o_ch, n), _F32),
    )


@functools.cache
def _contract_collapse(n: int, k_order: int, i_ch: int, o_ch: int, h_t: int,
                       b_n: int):
    ki = k_order * i_ch

    def body(th_ref, tl_ref, wh_ref, wl_ref, b_ref, out_ref):
        h = pl.program_id(1)
        xh = th_ref[...].reshape(ki, n)
        xl = tl_ref[...].reshape(ki, n)
        wh = wh_ref[0]
        wl = wl_ref[0]
        y = (jnp.dot(wh, xh, preferred_element_type=_F32)
             + jnp.dot(wl, xh, preferred_element_type=_F32)
             + jnp.dot(wh, xl, preferred_element_type=_F32))

        @pl.when(h == 0)
        def _():
            out_ref[...] = (y + float(h_t) * b_ref[...])[None]

        @pl.when(h > 0)
        def _():
            out_ref[...] = out_ref[...] + y[None]

    return pl.pallas_call(
        body,
        grid=(b_n, h_t),
        in_specs=[
            pl.BlockSpec((k_order, 1, 1, i_ch, n), lambda b, h: (0, h, b, 0, 0)),
            pl.BlockSpec((k_order, 1, 1, i_ch, n), lambda b, h: (0, h, b, 0, 0)),
            pl.BlockSpec((1, o_ch, ki), lambda b, h: (h, 0, 0)),
            pl.BlockSpec((1, o_ch, ki), lambda b, h: (h, 0, 0)),
            pl.BlockSpec((o_ch, 1), lambda b, h: (0, 0)),
        ],
        out_specs=pl.BlockSpec((1, o_ch, n), lambda b, h: (b, 0, 0)),
        out_shape=jax.ShapeDtypeStruct((b_n, o_ch, n), _F32),
    )


@functools.cache
def _pool(m_rows: int, n_in: int, n_out: int):
    dn = (((1,), (1,)), ((), ()))

    def body(a_ref, m_ref, o_ref):
        ah, al = _bsplit(a_ref[...])
        mh, ml = _bsplit(m_ref[...])
        o_ref[...] = (lax.dot_general(ah, mh, dn, preferred_element_type=_F32)
                      + lax.dot_general(al, mh, dn, preferred_element_type=_F32)
                      + lax.dot_general(ah, ml, dn, preferred_element_type=_F32))

    return pl.pallas_call(
        body,
        in_specs=[
            pl.BlockSpec((m_rows, n_in), lambda: (0, 0)),
            pl.BlockSpec((n_out, n_in), lambda: (0, 0)),
        ],
        out_specs=pl.BlockSpec((m_rows, n_out), lambda: (0, 0)),
        out_shape=jax.ShapeDtypeStruct((m_rows, n_out), _F32),
    )


@functools.cache
def _fc(b_n: int, feat: int, n_cls: int):
    dn = (((1,), (1,)), ((), ()))

    def body(z_ref, w_ref, b_ref, o_ref):
        zh, zl = _bsplit(z_ref[...])
        wh, wl = _bsplit(w_ref[...])
        logits = (lax.dot_general(zh, wh, dn, preferred_element_type=_F32)
                  + lax.dot_general(zl, wh, dn, preferred_element_type=_F32)
                  + lax.dot_general(zh, wl, dn, preferred_element_type=_F32)
                  ) + b_ref[...]
        m = jnp.max(logits, axis=1, keepdims=True)
        zz = logits - m
        o_ref[...] = zz - jnp.log(jnp.sum(jnp.exp(zz), axis=1, keepdims=True))

    return pl.pallas_call(
        body,
        in_specs=[
            pl.BlockSpec((b_n, feat), lambda: (0, 0)),
            pl.BlockSpec((n_cls, feat), lambda: (0, 0)),
            pl.BlockSpec((1, n_cls), lambda: (0, 0)),
        ],
        out_specs=pl.BlockSpec((b_n, n_cls), lambda: (0, 0)),
        out_shape=jax.ShapeDtypeStruct((b_n, n_cls), _F32),
    )


def kernel(x, edge_index_1, edge_weight_1, edge_index_2, edge_weight_2,
           edge_index_3, edge_weight_3, mapping_1, mapping_2, W1, b1, W2, b2,
           W3, b3, fc_w, fc_b):
    b_n, n1, h_t = x.shape
    k_order = W1.shape[0]
    n2, n3 = mapping_1.shape[0], mapping_2.shape[0]
    g1, g2, g3 = W1.shape[3], W2.shape[3], W3.shape[3]
    n_cls = fc_w.shape[0]

    at1, deg1 = _adj_builder(n1, edge_index_1.shape[1], 2)(edge_index_1,
                                                           edge_weight_1)
    at2, deg2 = _adj_builder(n2, edge_index_2.shape[1], 1)(edge_index_2,
                                                           edge_weight_2)
    at3, deg3 = _adj_builder(n3, edge_index_3.shape[1], 1)(edge_index_3,
                                                           edge_weight_3)

    xt1 = jnp.transpose(x, (2, 0, 1)).reshape(h_t * b_n, n1)
    th1, tl1 = _cheb(n1, h_t * b_n, k_order)(
        xt1, at1.reshape(n1, n1), deg1.reshape(1, n1))
    wt1h, wt1l = _bsplit(jnp.transpose(W1, (1, 3, 0, 2)).reshape(h_t, g1,
                                                                 k_order))
    o1 = _contract(n1, k_order, 1, g1, h_t, b_n, True)(
        th1.reshape(k_order, h_t, b_n, 1, n1),
        tl1.reshape(k_order, h_t, b_n, 1, n1), wt1h, wt1l, b1.reshape(g1, 1))
    x2 = _pool(h_t * b_n * g1, n1, n2)(o1.reshape(h_t * b_n * g1, n1),
                                       mapping_1)

    th2, tl2 = _cheb(n2, h_t * b_n * g1, k_order)(
        x2.reshape(h_t * b_n * g1, n2), at2.reshape(n2, n2),
        deg2.reshape(1, n2))
    wt2h, wt2l = _bsplit(jnp.transpose(W2, (1, 3, 0, 2)).reshape(
        h_t, g2, k_order * g1))
    o2 = _contract(n2, k_order, g1, g2, h_t, b_n, True)(
        th2.reshape(k_order, h_t, b_n, g1, n2),
        tl2.reshape(k_order, h_t, b_n, g1, n2), wt2h, wt2l, b2.reshape(g2, 1))
    x3 = _pool(h_t * b_n * g2, n2, n3)(o2.reshape(h_t * b_n * g2, n2),
                                       mapping_2)

    th3, tl3 = _cheb(n3, h_t * b_n * g2, k_order)(
        x3.reshape(h_t * b_n * g2, n3), at3.reshape(n3, n3),
        deg3.reshape(1, n3))
    wt3h, wt3l = _bsplit(jnp.transpose(W3, (1, 3, 0, 2)).reshape(
        h_t, g3, k_order * g2))
    y = _contract_collapse(n3, k_order, g2, g3, h_t, b_n)(
        th3.reshape(k_order, h_t, b_n, g2, n3),
        tl3.reshape(k_order, h_t, b_n, g2, n3), wt3h, wt3l, b3.reshape(g3, 1))

    z = jnp.transpose(y, (2, 1, 0)).reshape(b_n, n3 * g3)
    return _fc(b_n, n3 * g3, n_cls)(z, fc_w, fc_b.reshape(1, n_cls))


# speedup vs baseline: 212.8001x; 1.0750x over previous
import functools

import jax
import jax.numpy as jnp
from jax import lax
from jax.experimental import pallas as pl
from jax.experimental.pallas import tpu as pltpu
from jax.experimental.pallas import tpu_sc as plsc

_F32 = jnp.float32
_NC = 2
_NS = 16


@functools.cache
def _adj_builder(n: int, e: int, blocks_per_sc: int):
    shard = e // _NS
    n_chunks = shard // 128
    block_rows = n // (_NC * blocks_per_sc)
    blk_words = block_rows * n
    z_per_tile = blk_words // _NS
    zwords = min(8192, z_per_tile)
    nz = z_per_tile // zwords
    degq = n // _NS
    swords = min(8192, z_per_tile)
    ns = z_per_tile // swords

    mesh = plsc.VectorSubcoreMesh(core_axis_name="c", subcore_axis_name="s")

    @functools.partial(
        pl.kernel,
        out_type=(
            jax.ShapeDtypeStruct((n * n,), _F32),
            jax.ShapeDtypeStruct((n,), _F32),
        ),
        mesh=mesh,
        scratch_types=[
            pltpu.VMEM((shard,), jnp.int32),
            pltpu.VMEM((shard,), jnp.int32),
            pltpu.VMEM((shard,), _F32),
            pltpu.VMEM((n_chunks, 128), jnp.int32),
            pltpu.VMEM((n_chunks, 128), _F32),
            pltpu.VMEM((n_chunks, 128), jnp.int32),
            pltpu.VMEM((n_chunks, 128), _F32),
            pltpu.VMEM((zwords,), _F32),
            pltpu.VMEM((swords,), _F32),
            pltpu.VMEM_SHARED((blk_words,), _F32),
            pltpu.VMEM_SHARED((n,), _F32),
            pltpu.SemaphoreType.DMA,
        ],
    )
    def build(ei, ew, at_out, deg_out, r_buf, c_buf, w_buf, idx_buf, val_buf,
              didx_buf, dval_buf, zbuf, stage, sp_blk, deg_sp, sem):
        cid = lax.axis_index("c")
        sid = lax.axis_index("s")
        base = sid * shard
        lane = lax.iota(jnp.int32, 16)

        pltpu.async_copy(ei.at[0, pl.ds(base, shard)], r_buf, sem)
        pltpu.async_copy(ei.at[1, pl.ds(base, shard)], c_buf, sem)
        cw = pltpu.async_copy(ew.at[pl.ds(base, shard)], w_buf, sem)

        def _zb(i, carry):
            zbuf[pl.ds(i * 16, 16)] = jnp.zeros((16,), _F32)
            return carry
        lax.fori_loop(0, zwords // 16, _zb, 0)
        cw.wait()
        cw.wait()
        cw.wait()

        for b in range(blocks_per_sc):
            c0 = (cid * blocks_per_sc + b) * block_rows

            zd = [pltpu.async_copy(
                zbuf.at[pl.ds(0, zwords)],
                sp_blk.at[pl.ds(sid * z_per_tile + z * zwords, zwords)], sem)
                for z in range(nz)]
            if b == 0:
                @pl.when(cid == 0)
                def _():
                    pltpu.sync_copy(zbuf.at[pl.ds(0, degq)],
                                    deg_sp.at[pl.ds(sid * degq, degq)])
            for z in zd:
                z.wait()
            plsc.subcore_barrier()

            def _scan(j, carry):
                for g in range(8):
                    p0 = j * 128 + g * 16
                    sl = pl.ds(g * 16, 16)
                    psl = pl.ds(p0, 16)
                    r = r_buf[psl]
                    c = c_buf[psl]
                    w = w_buf[psl]
                    m = (c >= c0) & (c < c0 + block_rows)
                    fi = (c - c0) * n + r
                    pad = (p0 + lane) * _NS + sid
                    idx_buf[j, sl] = jnp.where(m, fi, pad)
                    val_buf[j, sl] = jnp.where(m, w, 0.0)
                    if b == 0:
                        didx_buf[j, sl] = r
                        dval_buf[j, sl] = w
                return carry
            lax.fori_loop(0, n_chunks, _scan, 0)

            sc_d = [pltpu.async_copy(val_buf.at[j], sp_blk.at[idx_buf.at[j]],
                                     sem, add=True) for j in range(n_chunks)]
            if b == 0:
                @pl.when(cid == 0)
                def _():
                    def _dchunk(j, carry):
                        pltpu.sync_copy(dval_buf.at[j],
                                        deg_sp.at[didx_buf.at[j]], add=True)
                        return carry
                    lax.fori_loop(0, n_chunks, _dchunk, 0)
            for d in sc_d:
                d.wait()
            plsc.subcore_barrier()

            if b == 0:
                @pl.when(cid == 0)
                def _():
                    pltpu.sync_copy(deg_sp.at[pl.ds(sid * degq, degq)],
                                    stage.at[pl.ds(0, degq)])
                    pltpu.sync_copy(stage.at[pl.ds(0, degq)],
                                    deg_out.at[pl.ds(sid * degq, degq)])
            for t in range(ns):
                off = sid * z_per_tile + t * swords
                pltpu.sync_copy(sp_blk.at[pl.ds(off, swords)], stage)
                pltpu.sync_copy(stage, at_out.at[pl.ds(c0 * n + off, swords)])
            plsc.subcore_barrier()

    return build


_BF = jnp.bfloat16


def _bsplit(v):
    hi = v.astype(_BF)
    lo = (v - hi.astype(_F32)).astype(_BF)
    return hi, lo


@functools.cache
def _cheb(n: int, r: int, k_order: int):
    def _dot3(uh, ul, ah, al):
        return (jnp.dot(uh, ah[...], preferred_element_type=_F32)
                + jnp.dot(ul, ah[...], preferred_element_type=_F32)
                + jnp.dot(uh, al[...], preferred_element_type=_F32))

    def body(x_ref, at_ref, deg_ref, hi_ref, ta, tb, dis, ah, al):
        k = pl.program_id(0)

        @pl.when(k == 0)
        def _():
            d = deg_ref[...]
            dis[...] = jnp.where(d > 0, lax.rsqrt(jnp.where(d > 0, d, 1.0)), 0.0)
            a_hi, a_lo = _bsplit(at_ref[...])
            ah[...] = a_hi
            al[...] = a_lo
            t0 = x_ref[...]
            ta[...] = t0
            tb[...] = jnp.zeros_like(t0)
            hi_ref[...] = t0.astype(_BF)[None]

        @pl.when((k % 2) == 1)
        def _():
            u = ta[...] * dis[...]
            uh, ul = _bsplit(u)
            p = _dot3(uh, ul, ah, al)
            coef = jnp.where(k == 1, -1.0, -2.0).astype(_F32)
            t = coef * (p * dis[...]) - tb[...]
            tb[...] = t
            hi_ref[...] = t.astype(_BF)[None]

        @pl.when((k >= 2) & ((k % 2) == 0))
        def _():
            u = tb[...] * dis[...]
            uh, ul = _bsplit(u)
            p = _dot3(uh, ul, ah, al)
            t = -2.0 * (p * dis[...]) - ta[...]
            ta[...] = t
            hi_ref[...] = t.astype(_BF)[None]

    return pl.pallas_call(
        body,
        grid=(k_order,),
        in_specs=[
            pl.BlockSpec((r, n), lambda k: (0, 0)),
            pl.BlockSpec((n, n), lambda k: (0, 0)),
            pl.BlockSpec((1, n), lambda k: (0, 0)),
        ],
        out_specs=pl.BlockSpec((1, r, n), lambda k: (k, 0, 0)),
        out_shape=jax.ShapeDtypeStruct((k_order, r, n), _BF),
        scratch_shapes=[
            pltpu.VMEM((r, n), _F32),
            pltpu.VMEM((r, n), _F32),
            pltpu.VMEM((1, n), _F32),
            pltpu.VMEM((n, n), _BF),
            pltpu.VMEM((n, n), _BF),
        ],
    )


@functools.cache
def _contract(n: int, k_order: int, i_ch: int, o_ch: int, h_t: int, b_n: int,
              relu: bool):
    ki = k_order * i_ch

    def body(th_ref, wh_ref, wl_ref, b_ref, out_ref):
        xh = th_ref[...].reshape(ki, n)
        wh = wh_ref[0]
        wl = wl_ref[0]
        y = (jnp.dot(wh, xh, preferred_element_type=_F32)
             + jnp.dot(wl, xh, preferred_element_type=_F32)) + b_ref[...]
        if relu:
            y = jnp.maximum(y, 0.0)
        out_ref[...] = y.reshape(1, 1, o_ch, n)

    return pl.pallas_call(
        body,
        grid=(b_n, h_t),
        in_specs=[
            pl.BlockSpec((k_order, 1, 1, i_ch, n), lambda b, h: (0, h, b, 0, 0)),
            pl.BlockSpec((1, o_ch, ki), lambda b, h: (h, 0, 0)),
            pl.BlockSpec((1, o_ch, ki), lambda b, h: (h, 0, 0)),
            pl.BlockSpec((o_ch, 1), lambda b, h: (0, 0)),
        ],
        out_specs=pl.BlockSpec((1, 1, o_ch, n), lambda b, h: (h, b, 0, 0)),
        out_shape=jax.ShapeDtypeStruct((h_t, b_n, o_ch, n), _F32),
    )


@functools.cache
def _contract_collapse(n: int, k_order: int, i_ch: int, o_ch: int, h_t: int,
                       b_n: int):
    ki = k_order * i_ch

    def body(th_ref, wh_ref, wl_ref, b_ref, out_ref):
        h = pl.program_id(1)
        xh = th_ref[...].reshape(ki, n)
        wh = wh_ref[0]
        wl = wl_ref[0]
        y = (jnp.dot(wh, xh, preferred_element_type=_F32)
             + jnp.dot(wl, xh, preferred_element_type=_F32))

        @pl.when(h == 0)
        def _():
            out_ref[...] = (y + float(h_t) * b_ref[...])[None]

        @pl.when(h > 0)
        def _():
            out_ref[...] = out_ref[...] + y[None]

    return pl.pallas_call(
        body,
        grid=(b_n, h_t),
        in_specs=[
            pl.BlockSpec((k_order, 1, 1, i_ch, n), lambda b, h: (0, h, b, 0, 0)),
            pl.BlockSpec((1, o_ch, ki), lambda b, h: (h, 0, 0)),
            pl.BlockSpec((1, o_ch, ki), lambda b, h: (h, 0, 0)),
            pl.BlockSpec((o_ch, 1), lambda b, h: (0, 0)),
        ],
        out_specs=pl.BlockSpec((1, o_ch, n), lambda b, h: (b, 0, 0)),
        out_shape=jax.ShapeDtypeStruct((b_n, o_ch, n), _F32),
    )


@functools.cache
def _pool(m_rows: int, n_in: int, n_out: int):
    dn = (((1,), (1,)), ((), ()))

    def body(a_ref, m_ref, o_ref):
        ah = a_ref[...].astype(_BF)
        mh, ml = _bsplit(m_ref[...])
        o_ref[...] = (lax.dot_general(ah, mh, dn, preferred_element_type=_F32)
                      + lax.dot_general(ah, ml, dn, preferred_element_type=_F32))

    return pl.pallas_call(
        body,
        in_specs=[
            pl.BlockSpec((m_rows, n_in), lambda: (0, 0)),
            pl.BlockSpec((n_out, n_in), lambda: (0, 0)),
        ],
        out_specs=pl.BlockSpec((m_rows, n_out), lambda: (0, 0)),
        out_shape=jax.ShapeDtypeStruct((m_rows, n_out), _F32),
    )


@functools.cache
def _fc(b_n: int, feat: int, n_cls: int):
    dn = (((1,), (1,)), ((), ()))

    def body(z_ref, w_ref, b_ref, o_ref):
        zh, zl = _bsplit(z_ref[...])
        wh, wl = _bsplit(w_ref[...])
        logits = (lax.dot_general(zh, wh, dn, preferred_element_type=_F32)
                  + lax.dot_general(zl, wh, dn, preferred_element_type=_F32)
                  + lax.dot_general(zh, wl, dn, preferred_element_type=_F32)
                  ) + b_ref[...]
        m = jnp.max(logits, axis=1, keepdims=True)
        zz = logits - m
        o_ref[...] = zz - jnp.log(jnp.sum(jnp.exp(zz), axis=1, keepdims=True))

    return pl.pallas_call(
        body,
        in_specs=[
            pl.BlockSpec((b_n, feat), lambda: (0, 0)),
            pl.BlockSpec((n_cls, feat), lambda: (0, 0)),
            pl.BlockSpec((1, n_cls), lambda: (0, 0)),
        ],
        out_specs=pl.BlockSpec((b_n, n_cls), lambda: (0, 0)),
        out_shape=jax.ShapeDtypeStruct((b_n, n_cls), _F32),
    )


def kernel(x, edge_index_1, edge_weight_1, edge_index_2, edge_weight_2,
           edge_index_3, edge_weight_3, mapping_1, mapping_2, W1, b1, W2, b2,
           W3, b3, fc_w, fc_b):
    b_n, n1, h_t = x.shape
    k_order = W1.shape[0]
    n2, n3 = mapping_1.shape[0], mapping_2.shape[0]
    g1, g2, g3 = W1.shape[3], W2.shape[3], W3.shape[3]
    n_cls = fc_w.shape[0]

    at1, deg1 = _adj_builder(n1, edge_index_1.shape[1], 2)(edge_index_1,
                                                           edge_weight_1)
    at2, deg2 = _adj_builder(n2, edge_index_2.shape[1], 1)(edge_index_2,
                                                           edge_weight_2)
    at3, deg3 = _adj_builder(n3, edge_index_3.shape[1], 1)(edge_index_3,
                                                           edge_weight_3)

    xt1 = jnp.transpose(x, (2, 0, 1)).reshape(h_t * b_n, n1)
    th1 = _cheb(n1, h_t * b_n, k_order)(
        xt1, at1.reshape(n1, n1), deg1.reshape(1, n1))
    wt1h, wt1l = _bsplit(jnp.transpose(W1, (1, 3, 0, 2)).reshape(h_t, g1,
                                                                 k_order))
    o1 = _contract(n1, k_order, 1, g1, h_t, b_n, True)(
        th1.reshape(k_order, h_t, b_n, 1, n1), wt1h, wt1l, b1.reshape(g1, 1))
    x2 = _pool(h_t * b_n * g1, n1, n2)(o1.reshape(h_t * b_n * g1, n1),
                                       mapping_1)

    th2 = _cheb(n2, h_t * b_n * g1, k_order)(
        x2.reshape(h_t * b_n * g1, n2), at2.reshape(n2, n2),
        deg2.reshape(1, n2))
    wt2h, wt2l = _bsplit(jnp.transpose(W2, (1, 3, 0, 2)).reshape(
        h_t, g2, k_order * g1))
    o2 = _contract(n2, k_order, g1, g2, h_t, b_n, True)(
        th2.reshape(k_order, h_t, b_n, g1, n2), wt2h, wt2l, b2.reshape(g2, 1))
    x3 = _pool(h_t * b_n * g2, n2, n3)(o2.reshape(h_t * b_n * g2, n2),
                                       mapping_2)

    th3 = _cheb(n3, h_t * b_n * g2, k_order)(
        x3.reshape(h_t * b_n * g2, n3), at3.reshape(n3, n3),
        deg3.reshape(1, n3))
    wt3h, wt3l = _bsplit(jnp.transpose(W3, (1, 3, 0, 2)).reshape(
        h_t, g3, k_order * g2))
    y = _contract_collapse(n3, k_order, g2, g3, h_t, b_n)(
        th3.reshape(k_order, h_t, b_n, g2, n3), wt3h, wt3l, b3.reshape(g3, 1))

    z = jnp.transpose(y, (2, 1, 0)).reshape(b_n, n3 * g3)
    return _fc(b_n, n3 * g3, n_cls)(z, fc_w, fc_b.reshape(1, n_cls))


# speedup vs baseline: 249.9930x; 1.1748x over previous
import functools

import jax
import jax.numpy as jnp
from jax import lax
from jax.experimental import pallas as pl
from jax.experimental.pallas import tpu as pltpu
from jax.experimental.pallas import tpu_sc as plsc

_F32 = jnp.float32
_NC = 2
_NS = 16


@functools.cache
def _adj_builder(n: int, e: int, blocks_per_sc: int):
    shard = e // _NS
    n_chunks = shard // 128
    block_rows = n // (_NC * blocks_per_sc)
    blk_words = block_rows * n
    z_per_tile = blk_words // _NS
    zwords = min(8192, z_per_tile)
    nz = z_per_tile // zwords
    degq = n // _NS
    swords = min(8192, z_per_tile)
    ns = z_per_tile // swords

    mesh = plsc.VectorSubcoreMesh(core_axis_name="c", subcore_axis_name="s")

    @functools.partial(
        pl.kernel,
        out_type=(
            jax.ShapeDtypeStruct((n * n,), _F32),
            jax.ShapeDtypeStruct((n,), _F32),
        ),
        mesh=mesh,
        scratch_types=[
            pltpu.VMEM((shard,), jnp.int32),
            pltpu.VMEM((shard,), jnp.int32),
            pltpu.VMEM((shard,), _F32),
            pltpu.VMEM((n_chunks, 128), jnp.int32),
            pltpu.VMEM((n_chunks, 128), _F32),
            pltpu.VMEM((n_chunks, 128), jnp.int32),
            pltpu.VMEM((n_chunks, 128), _F32),
            pltpu.VMEM((zwords,), _F32),
            pltpu.VMEM((swords,), _F32),
            pltpu.VMEM_SHARED((blk_words,), _F32),
            pltpu.VMEM_SHARED((n,), _F32),
            pltpu.SemaphoreType.DMA,
        ],
    )
    def build(ei, ew, at_out, deg_out, r_buf, c_buf, w_buf, idx_buf, val_buf,
              didx_buf, dval_buf, zbuf, stage, sp_blk, deg_sp, sem):
        cid = lax.axis_index("c")
        sid = lax.axis_index("s")
        base = sid * shard
        lane = lax.iota(jnp.int32, 16)

        pltpu.async_copy(ei.at[0, pl.ds(base, shard)], r_buf, sem)
        pltpu.async_copy(ei.at[1, pl.ds(base, shard)], c_buf, sem)
        cw = pltpu.async_copy(ew.at[pl.ds(base, shard)], w_buf, sem)

        def _zb(i, carry):
            zbuf[pl.ds(i * 16, 16)] = jnp.zeros((16,), _F32)
            return carry
        lax.fori_loop(0, zwords // 16, _zb, 0)
        cw.wait()
        cw.wait()
        cw.wait()

        for b in range(blocks_per_sc):
            c0 = (cid * blocks_per_sc + b) * block_rows

            zd = [pltpu.async_copy(
                zbuf.at[pl.ds(0, zwords)],
                sp_blk.at[pl.ds(sid * z_per_tile + z * zwords, zwords)], sem)
                for z in range(nz)]
            if b == 0:
                @pl.when(cid == 0)
                def _():
                    pltpu.sync_copy(zbuf.at[pl.ds(0, degq)],
                                    deg_sp.at[pl.ds(sid * degq, degq)])
            for z in zd:
                z.wait()
            plsc.subcore_barrier()

            def _scan(j, carry):
                for g in range(8):
                    p0 = j * 128 + g * 16
                    sl = pl.ds(g * 16, 16)
                    psl = pl.ds(p0, 16)
                    r = r_buf[psl]
                    c = c_buf[psl]
                    w = w_buf[psl]
                    m = (c >= c0) & (c < c0 + block_rows)
                    fi = (c - c0) * n + r
                    pad = (p0 + lane) * _NS + sid
                    idx_buf[j, sl] = jnp.where(m, fi, pad)
                    val_buf[j, sl] = jnp.where(m, w, 0.0)
                    if b == 0:
                        didx_buf[j, sl] = r
                        dval_buf[j, sl] = w
                return carry
            lax.fori_loop(0, n_chunks, _scan, 0)

            sc_d = [pltpu.async_copy(val_buf.at[j], sp_blk.at[idx_buf.at[j]],
                                     sem, add=True) for j in range(n_chunks)]
            if b == 0:
                @pl.when(cid == 0)
                def _():
                    def _dchunk(j, carry):
                        pltpu.sync_copy(dval_buf.at[j],
                                        deg_sp.at[didx_buf.at[j]], add=True)
                        return carry
                    lax.fori_loop(0, n_chunks, _dchunk, 0)
            for d in sc_d:
                d.wait()
            plsc.subcore_barrier()

            if b == 0:
                @pl.when(cid == 0)
                def _():
                    pltpu.sync_copy(deg_sp.at[pl.ds(sid * degq, degq)],
                                    stage.at[pl.ds(0, degq)])
                    pltpu.sync_copy(stage.at[pl.ds(0, degq)],
                                    deg_out.at[pl.ds(sid * degq, degq)])
            for t in range(ns):
                off = sid * z_per_tile + t * swords
                pltpu.sync_copy(sp_blk.at[pl.ds(off, swords)], stage)
                pltpu.sync_copy(stage, at_out.at[pl.ds(c0 * n + off, swords)])
            plsc.subcore_barrier()

    return build


_BF = jnp.bfloat16


def _bsplit(v):
    hi = v.astype(_BF)
    lo = (v - hi.astype(_F32)).astype(_BF)
    return hi, lo


@functools.cache
def _cheb(n: int, r: int, k_order: int):
    def _dot3(uh, ul, ah, al):
        return (jnp.dot(uh, ah[...], preferred_element_type=_F32)
                + jnp.dot(ul, ah[...], preferred_element_type=_F32)
                + jnp.dot(uh, al[...], preferred_element_type=_F32))

    def body(x_ref, at_ref, deg_ref, hi_ref, ta, tb, dis, ah, al):
        k = pl.program_id(0)

        @pl.when(k == 0)
        def _():
            d = deg_ref[...]
            dis[...] = jnp.where(d > 0, lax.rsqrt(jnp.where(d > 0, d, 1.0)), 0.0)
            a_hi, a_lo = _bsplit(at_ref[...])
            ah[...] = a_hi
            al[...] = a_lo
            t0 = x_ref[...]
            ta[...] = t0
            tb[...] = jnp.zeros_like(t0)
            hi_ref[...] = t0.astype(_BF)[None]

        @pl.when((k % 2) == 1)
        def _():
            u = ta[...] * dis[...]
            uh, ul = _bsplit(u)
            p = _dot3(uh, ul, ah, al)
            coef = jnp.where(k == 1, -1.0, -2.0).astype(_F32)
            t = coef * (p * dis[...]) - tb[...]
            tb[...] = t
            hi_ref[...] = t.astype(_BF)[None]

        @pl.when((k >= 2) & ((k % 2) == 0))
        def _():
            u = tb[...] * dis[...]
            uh, ul = _bsplit(u)
            p = _dot3(uh, ul, ah, al)
            t = -2.0 * (p * dis[...]) - ta[...]
            ta[...] = t
            hi_ref[...] = t.astype(_BF)[None]

    return pl.pallas_call(
        body,
        grid=(k_order,),
        in_specs=[
            pl.BlockSpec((r, n), lambda k: (0, 0)),
            pl.BlockSpec((n, n), lambda k: (0, 0)),
            pl.BlockSpec((1, n), lambda k: (0, 0)),
        ],
        out_specs=pl.BlockSpec((1, r, n), lambda k: (k, 0, 0)),
        out_shape=jax.ShapeDtypeStruct((k_order, r, n), _BF),
        scratch_shapes=[
            pltpu.VMEM((r, n), _F32),
            pltpu.VMEM((r, n), _F32),
            pltpu.VMEM((1, n), _F32),
            pltpu.VMEM((n, n), _BF),
            pltpu.VMEM((n, n), _BF),
        ],
    )


@functools.cache
def _contract_pool(n: int, k_order: int, i_ch: int, o_ch: int, h_t: int,
                   b_n: int, n_out: int):
    ki = k_order * i_ch
    dn = (((1,), (1,)), ((), ()))

    def body(t_ref, wh_ref, wl_ref, b_ref, mh_ref, ml_ref, o_ref, acc):
        for h in range(h_t):
            for b in range(b_n):
                xh = t_ref[:, h, b].reshape(ki, n)
                y = (jnp.dot(wh_ref[h], xh, preferred_element_type=_F32)
                     + jnp.dot(wl_ref[h], xh, preferred_element_type=_F32)
                     ) + b_ref[...]
                acc[pl.ds((h * b_n + b) * o_ch, o_ch), :] = jnp.maximum(y, 0.0)
        ah = acc[...].astype(_BF)
        o_ref[...] = (lax.dot_general(ah, mh_ref[...], dn,
                                      preferred_element_type=_F32)
                      + lax.dot_general(ah, ml_ref[...], dn,
                                        preferred_element_type=_F32))

    return pl.pallas_call(
        body,
        in_specs=[
            pl.BlockSpec((k_order, h_t, b_n, i_ch, n),
                         lambda: (0, 0, 0, 0, 0)),
            pl.BlockSpec((h_t, o_ch, ki), lambda: (0, 0, 0)),
            pl.BlockSpec((h_t, o_ch, ki), lambda: (0, 0, 0)),
            pl.BlockSpec((o_ch, 1), lambda: (0, 0)),
            pl.BlockSpec((n_out, n), lambda: (0, 0)),
            pl.BlockSpec((n_out, n), lambda: (0, 0)),
        ],
        out_specs=pl.BlockSpec((h_t * b_n * o_ch, n_out), lambda: (0, 0)),
        out_shape=jax.ShapeDtypeStruct((h_t * b_n * o_ch, n_out), _F32),
        scratch_shapes=[pltpu.VMEM((h_t * b_n * o_ch, n), _F32)],
    )


@functools.cache
def _contract_collapse(n: int, k_order: int, i_ch: int, o_ch: int, h_t: int,
                       b_n: int):
    ki = k_order * i_ch

    def body(t_ref, wh_ref, wl_ref, b_ref, o_ref):
        for b in range(b_n):
            acc = float(h_t) * jnp.broadcast_to(b_ref[...], (o_ch, n))
            for h in range(h_t):
                xh = t_ref[:, h, b].reshape(ki, n)
                acc = (acc + jnp.dot(wh_ref[h], xh, preferred_element_type=_F32)
                       + jnp.dot(wl_ref[h], xh, preferred_element_type=_F32))
            o_ref[b] = acc

    return pl.pallas_call(
        body,
        in_specs=[
            pl.BlockSpec((k_order, h_t, b_n, i_ch, n),
                         lambda: (0, 0, 0, 0, 0)),
            pl.BlockSpec((h_t, o_ch, ki), lambda: (0, 0, 0)),
            pl.BlockSpec((h_t, o_ch, ki), lambda: (0, 0, 0)),
            pl.BlockSpec((o_ch, 1), lambda: (0, 0)),
        ],
        out_specs=pl.BlockSpec((b_n, o_ch, n), lambda: (0, 0, 0)),
        out_shape=jax.ShapeDtypeStruct((b_n, o_ch, n), _F32),
    )


@functools.cache
def _fc(b_n: int, feat: int, n_cls: int):
    dn = (((1,), (1,)), ((), ()))

    def body(z_ref, w_ref, b_ref, o_ref):
        zh, zl = _bsplit(z_ref[...])
        wh, wl = _bsplit(w_ref[...])
        logits = (lax.dot_general(zh, wh, dn, preferred_element_type=_F32)
                  + lax.dot_general(zl, wh, dn, preferred_element_type=_F32)
                  + lax.dot_general(zh, wl, dn, preferred_element_type=_F32)
                  ) + b_ref[...]
        m = jnp.max(logits, axis=1, keepdims=True)
        zz = logits - m
        o_ref[...] = zz - jnp.log(jnp.sum(jnp.exp(zz), axis=1, keepdims=True))

    return pl.pallas_call(
        body,
        in_specs=[
            pl.BlockSpec((b_n, feat), lambda: (0, 0)),
            pl.BlockSpec((n_cls, feat), lambda: (0, 0)),
            pl.BlockSpec((1, n_cls), lambda: (0, 0)),
        ],
        out_specs=pl.BlockSpec((b_n, n_cls), lambda: (0, 0)),
        out_shape=jax.ShapeDtypeStruct((b_n, n_cls), _F32),
    )


def kernel(x, edge_index_1, edge_weight_1, edge_index_2, edge_weight_2,
           edge_index_3, edge_weight_3, mapping_1, mapping_2, W1, b1, W2, b2,
           W3, b3, fc_w, fc_b):
    b_n, n1, h_t = x.shape
    k_order = W1.shape[0]
    n2, n3 = mapping_1.shape[0], mapping_2.shape[0]
    g1, g2, g3 = W1.shape[3], W2.shape[3], W3.shape[3]
    n_cls = fc_w.shape[0]

    at1, deg1 = _adj_builder(n1, edge_index_1.shape[1], 2)(edge_index_1,
                                                           edge_weight_1)
    at2, deg2 = _adj_builder(n2, edge_index_2.shape[1], 1)(edge_index_2,
                                                           edge_weight_2)
    at3, deg3 = _adj_builder(n3, edge_index_3.shape[1], 1)(edge_index_3,
                                                           edge_weight_3)

    m1h, m1l = _bsplit(mapping_1)
    m2h, m2l = _bsplit(mapping_2)
    xt1 = jnp.transpose(x, (2, 0, 1)).reshape(h_t * b_n, n1)
    th1 = _cheb(n1, h_t * b_n, k_order)(
        xt1, at1.reshape(n1, n1), deg1.reshape(1, n1))
    wt1h, wt1l = _bsplit(jnp.transpose(W1, (1, 3, 0, 2)).reshape(h_t, g1,
                                                                 k_order))
    x2 = _contract_pool(n1, k_order, 1, g1, h_t, b_n, n2)(
        th1.reshape(k_order, h_t, b_n, 1, n1), wt1h, wt1l, b1.reshape(g1, 1),
        m1h, m1l)

    th2 = _cheb(n2, h_t * b_n * g1, k_order)(
        x2, at2.reshape(n2, n2), deg2.reshape(1, n2))
    wt2h, wt2l = _bsplit(jnp.transpose(W2, (1, 3, 0, 2)).reshape(
        h_t, g2, k_order * g1))
    x3 = _contract_pool(n2, k_order, g1, g2, h_t, b_n, n3)(
        th2.reshape(k_order, h_t, b_n, g1, n2), wt2h, wt2l, b2.reshape(g2, 1),
        m2h, m2l)

    th3 = _cheb(n3, h_t * b_n * g2, k_order)(
        x3, at3.reshape(n3, n3), deg3.reshape(1, n3))
    wt3h, wt3l = _bsplit(jnp.transpose(W3, (1, 3, 0, 2)).reshape(
        h_t, g3, k_order * g2))
    y = _contract_collapse(n3, k_order, g2, g3, h_t, b_n)(
        th3.reshape(k_order, h_t, b_n, g2, n3), wt3h, wt3l, b3.reshape(g3, 1))

    z = jnp.transpose(y, (2, 1, 0)).reshape(b_n, n3 * g3)
    return _fc(b_n, n3 * g3, n_cls)(z, fc_w, fc_b.reshape(1, n_cls))
